# baseline (speedup 1.0000x reference)
import jax
import jax.numpy as jnp
from jax import lax
from jax.experimental import pallas as pl

_TOTAL_LABELS = 81
_TOTAL_POS = 500
_TOTAL_NEG = 1000
_POS_TH = 0.5
_SEED = 42
_SCALE = 8192
_N_ROI = 5000
_N_GT = 100
_CHUNK = 1000


def _stage_a(pk_ref, gtt_ref, lab_ref, lout_ref, aux_ref):
    pk = pk_ref[0]
    gtt = gtt_ref[0]
    lab = lab_ref[0]

    r_y1 = pk[:, 0:1]
    r_x1 = pk[:, 1:2]
    r_y2 = pk[:, 2:3]
    r_x2 = pk[:, 3:4]
    rpos = pk[:, 4:5].astype(jnp.int32)
    rneg = pk[:, 5:6].astype(jnp.int32)

    g_y1 = gtt[0:1, :]
    g_x1 = gtt[1:2, :]
    g_y2 = gtt[2:3, :]
    g_x2 = gtt[3:4, :]

    gt_area = (g_y2 - g_y1) * (g_x2 - g_x1)
    bbox_area = (r_y2 - r_y1) * (r_x2 - r_x1)
    x_top = jnp.maximum(r_x1, g_x1)
    y_top = jnp.maximum(r_y1, g_y1)
    x_bot = jnp.minimum(r_x2, g_x2)
    y_bot = jnp.minimum(r_y2, g_y2)
    inter = jnp.maximum(x_bot - x_top, 0.0) * jnp.maximum(y_bot - y_top, 0.0)
    union = bbox_area + gt_area - inter
    iou = inter / union

    merged = jnp.max(iou, axis=1, keepdims=True)
    iota_g = lax.broadcasted_iota(jnp.int32, (_N_ROI, _N_GT), 1)
    idx = jnp.min(jnp.where(iou == merged, iota_g, _N_GT), axis=1, keepdims=True)

    hit = iota_g == idx
    hitf = hit.astype(jnp.float32)
    gb_y1 = jnp.sum(hitf * g_y1, axis=1, keepdims=True)
    gb_x1 = jnp.sum(hitf * g_x1, axis=1, keepdims=True)
    gb_y2 = jnp.sum(hitf * g_y2, axis=1, keepdims=True)
    gb_x2 = jnp.sum(hitf * g_x2, axis=1, keepdims=True)
    glab = jnp.sum(jnp.where(hit, lab, 0), axis=1, keepdims=True)

    pos_mask = merged > _POS_TH
    neg_mask = jnp.logical_and(merged < _POS_TH, merged >= 0.0)

    iota_r = lax.broadcasted_iota(jnp.int32, (_N_ROI, 1), 0)
    tie = (_SCALE - 1) - iota_r
    c_pos = jnp.where(pos_mask, rpos, 0) * _SCALE + tie
    c_neg = jnp.where(neg_mask, rneg, 0) * _SCALE + tie
    c_both = jnp.concatenate([c_pos, c_neg], axis=1)
    col2 = lax.broadcasted_iota(jnp.int32, (1, 2), 1)
    kvec = jnp.where(col2 == 0, _TOTAL_POS, _TOTAL_NEG)

    def bs_body(_, carry):
        lo, hi = carry
        mid = (lo + hi + 1) // 2
        cnt = jnp.sum((c_both >= mid).astype(jnp.int32), axis=0, keepdims=True)
        ge = cnt >= kvec
        return jnp.where(ge, mid, lo), jnp.where(ge, hi, mid - 1)

    lo0 = jnp.zeros((1, 2), jnp.int32)
    hi0 = jnp.full((1, 2), (1 << 27) - 1, jnp.int32)
    thr, _ = lax.fori_loop(0, 27, bs_body, (lo0, hi0))
    pos_sel = jnp.logical_and(pos_mask, c_pos >= thr[0:1, 0:1])
    neg_sel = jnp.logical_and(neg_mask, c_neg >= thr[0:1, 1:2])

    zero = jnp.zeros_like(gb_y1)
    e_y1 = jnp.where(pos_sel, gb_y1, zero)
    e_x1 = jnp.where(pos_sel, gb_x1, zero)
    e_y2 = jnp.where(pos_sel, gb_y2, zero)
    e_x2 = jnp.where(pos_sel, gb_x2, zero)

    bw = r_x2 - r_x1
    bh = r_y2 - r_y1
    bcx = r_x1 + 0.5 * bw
    bcy = r_y1 + 0.5 * bh
    gw = e_x2 - e_x1
    gh = e_y2 - e_y1
    gcx = e_x1 + 0.5 * gw
    gcy = e_y1 + 0.5 * gh
    bw = jnp.where(bw == 0.0, 1e-3, bw)
    bh = jnp.where(bh == 0.0, 1e-3, bh)
    wz = gw == 0.0
    hz = gh == 0.0
    d_y = jnp.where(hz, zero, (gcy - bcy) / bh) / 0.1
    d_x = jnp.where(wz, zero, (gcx - bcx) / bw) / 0.1
    d_h = jnp.where(hz, zero, jnp.log(jnp.where(hz, 1.0, gh) / bh)) / 0.2
    d_w = jnp.where(wz, zero, jnp.log(jnp.where(wz, 1.0, gw) / bw)) / 0.2

    final_lab = jnp.where(pos_sel, glab, -1) + jnp.where(neg_sel, 1, 0)

    iota_l = lax.broadcasted_iota(jnp.int32, (_N_ROI, _TOTAL_LABELS), 1)
    lout_ref[0] = (iota_l == final_lab).astype(jnp.float32)

    aux_ref[0] = jnp.concatenate(
        [d_y, d_x, d_h, d_w, final_lab.astype(jnp.float32)], axis=1)


def _stage_b(aux_ref, cc_ref, dout_ref):
    aux = aux_ref[0]
    lbl_c = cc_ref[0, 0:1, :]
    comp = cc_ref[0, 1:2, :]
    d_y = aux[:, 0:1]
    d_x = aux[:, 1:2]
    d_h = aux[:, 2:3]
    d_w = aux[:, 3:4]
    fl = aux[:, 4:5].astype(jnp.int32)
    onehot_c = (lbl_c == fl).astype(jnp.float32)
    dval = jnp.where(comp == 0, d_y,
                     jnp.where(comp == 1, d_x,
                               jnp.where(comp == 2, d_h, d_w)))
    dout_ref[0] = onehot_c * dval


def _run(pk, gtt, lab3, interpret=False):
    b = pk.shape[0]
    labels, aux = pl.pallas_call(
        _stage_a,
        grid=(b,),
        in_specs=[
            pl.BlockSpec((1, _N_ROI, 6), lambda i: (i, 0, 0)),
            pl.BlockSpec((1, 4, _N_GT), lambda i: (i, 0, 0)),
            pl.BlockSpec((1, 1, _N_GT), lambda i: (i, 0, 0)),
        ],
        out_specs=[
            pl.BlockSpec((1, _N_ROI, _TOTAL_LABELS), lambda i: (i, 0, 0)),
            pl.BlockSpec((1, _N_ROI, 5), lambda i: (i, 0, 0)),
        ],
        out_shape=[
            jax.ShapeDtypeStruct((b, _N_ROI, _TOTAL_LABELS), jnp.float32),
            jax.ShapeDtypeStruct((b, _N_ROI, 5), jnp.float32),
        ],
        interpret=interpret,
    )(pk, gtt, lab3)

    iota_c = jnp.arange(_TOTAL_LABELS * 4, dtype=jnp.int32)
    cc = jnp.stack([iota_c // 4, iota_c % 4])[None]

    d_expanded = pl.pallas_call(
        _stage_b,
        grid=(b, _N_ROI // _CHUNK),
        in_specs=[
            pl.BlockSpec((1, _CHUNK, 5), lambda i, j: (i, j, 0)),
            pl.BlockSpec((1, 2, _TOTAL_LABELS * 4), lambda i, j: (0, 0, 0)),
        ],
        out_specs=pl.BlockSpec((1, _CHUNK, _TOTAL_LABELS * 4), lambda i, j: (i, j, 0)),
        out_shape=jax.ShapeDtypeStruct((b, _N_ROI, _TOTAL_LABELS * 4), jnp.float32),
        interpret=interpret,
    )(aux, cc)
    return d_expanded, labels


def kernel(roi_bboxes, gt_boxes, gt_labels):
    b, n = roi_bboxes.shape[0], roi_bboxes.shape[1]
    gtt = jnp.transpose(gt_boxes, (0, 2, 1))
    lab3 = gt_labels[:, None, :]
    rpos = jax.random.randint(jax.random.key(_SEED), (b, n), 1,
                              _TOTAL_POS * 10, dtype=jnp.int32)[..., None]
    rneg = jax.random.randint(jax.random.key(_SEED + 1), (b, n), 1,
                              _TOTAL_NEG * 10, dtype=jnp.int32)[..., None]
    pk = jnp.concatenate(
        [roi_bboxes, rpos.astype(jnp.float32), rneg.astype(jnp.float32)],
        axis=-1)
    d_expanded, labels = _run(pk, gtt, lab3)
    deltas = d_expanded.reshape(b, n * _TOTAL_LABELS, 4)
    return deltas, labels


# speedup vs baseline: 2.8052x; 2.8052x over previous
import jax
import jax.numpy as jnp
from jax import lax
from jax.experimental import pallas as pl

_TOTAL_LABELS = 81
_TOTAL_POS = 500
_TOTAL_NEG = 1000
_POS_TH = 0.5
_SEED = 42
_SCALE = 8192
_N_ROI = 5000
_N_GT = 100
_CHUNK = 1000


def _stage_a(pk_ref, gtt_ref, lab_ref, lout_ref, aux_ref):
    pk = pk_ref[0]
    gtt = gtt_ref[0]
    lab = lab_ref[0]

    r_y1 = pk[:, 0:1]
    r_x1 = pk[:, 1:2]
    r_y2 = pk[:, 2:3]
    r_x2 = pk[:, 3:4]
    rpos = pk[:, 4:5].astype(jnp.int32)
    rneg = pk[:, 5:6].astype(jnp.int32)

    g_y1 = gtt[0:1, :]
    g_x1 = gtt[1:2, :]
    g_y2 = gtt[2:3, :]
    g_x2 = gtt[3:4, :]

    gt_area = (g_y2 - g_y1) * (g_x2 - g_x1)
    bbox_area = (r_y2 - r_y1) * (r_x2 - r_x1)
    x_top = jnp.maximum(r_x1, g_x1)
    y_top = jnp.maximum(r_y1, g_y1)
    x_bot = jnp.minimum(r_x2, g_x2)
    y_bot = jnp.minimum(r_y2, g_y2)
    inter = jnp.maximum(x_bot - x_top, 0.0) * jnp.maximum(y_bot - y_top, 0.0)
    union = bbox_area + gt_area - inter
    iou = inter / union

    merged = jnp.max(iou, axis=1, keepdims=True)
    iota_g = lax.broadcasted_iota(jnp.int32, (_N_ROI, _N_GT), 1)
    idx = jnp.min(jnp.where(iou == merged, iota_g, _N_GT), axis=1, keepdims=True)

    hit = iota_g == idx
    hitf = hit.astype(jnp.float32)
    gb_y1 = jnp.sum(hitf * g_y1, axis=1, keepdims=True)
    gb_x1 = jnp.sum(hitf * g_x1, axis=1, keepdims=True)
    gb_y2 = jnp.sum(hitf * g_y2, axis=1, keepdims=True)
    gb_x2 = jnp.sum(hitf * g_x2, axis=1, keepdims=True)
    glab = jnp.sum(jnp.where(hit, lab, 0), axis=1, keepdims=True)

    pos_mask = merged > _POS_TH
    neg_mask = jnp.logical_and(merged < _POS_TH, merged >= 0.0)

    iota_r = lax.broadcasted_iota(jnp.int32, (_N_ROI, 1), 0)
    tie = (_SCALE - 1) - iota_r
    c_pos = jnp.where(pos_mask, rpos, 0) * _SCALE + tie
    c_neg = jnp.where(neg_mask, rneg, 0) * _SCALE + tie
    c_both = jnp.concatenate([c_pos, c_neg], axis=1)
    col2 = lax.broadcasted_iota(jnp.int32, (1, 2), 1)
    kvec = jnp.where(col2 == 0, _TOTAL_POS, _TOTAL_NEG)

    def bs_body(_, carry):
        lo, hi = carry
        mid = (lo + hi + 1) // 2
        cnt = jnp.sum((c_both >= mid).astype(jnp.int32), axis=0, keepdims=True)
        ge = cnt >= kvec
        return jnp.where(ge, mid, lo), jnp.where(ge, hi, mid - 1)

    lo0 = jnp.zeros((1, 2), jnp.int32)
    hi0 = jnp.full((1, 2), (1 << 27) - 1, jnp.int32)
    thr, _ = lax.fori_loop(0, 27, bs_body, (lo0, hi0))
    pos_sel = jnp.logical_and(pos_mask, c_pos >= thr[0:1, 0:1])
    neg_sel = jnp.logical_and(neg_mask, c_neg >= thr[0:1, 1:2])

    zero = jnp.zeros_like(gb_y1)
    e_y1 = jnp.where(pos_sel, gb_y1, zero)
    e_x1 = jnp.where(pos_sel, gb_x1, zero)
    e_y2 = jnp.where(pos_sel, gb_y2, zero)
    e_x2 = jnp.where(pos_sel, gb_x2, zero)

    bw = r_x2 - r_x1
    bh = r_y2 - r_y1
    bcx = r_x1 + 0.5 * bw
    bcy = r_y1 + 0.5 * bh
    gw = e_x2 - e_x1
    gh = e_y2 - e_y1
    gcx = e_x1 + 0.5 * gw
    gcy = e_y1 + 0.5 * gh
    bw = jnp.where(bw == 0.0, 1e-3, bw)
    bh = jnp.where(bh == 0.0, 1e-3, bh)
    wz = gw == 0.0
    hz = gh == 0.0
    d_y = jnp.where(hz, zero, (gcy - bcy) / bh) / 0.1
    d_x = jnp.where(wz, zero, (gcx - bcx) / bw) / 0.1
    d_h = jnp.where(hz, zero, jnp.log(jnp.where(hz, 1.0, gh) / bh)) / 0.2
    d_w = jnp.where(wz, zero, jnp.log(jnp.where(wz, 1.0, gw) / bw)) / 0.2

    final_lab = jnp.where(pos_sel, glab, -1) + jnp.where(neg_sel, 1, 0)

    iota_l = lax.broadcasted_iota(jnp.int32, (_N_ROI, _TOTAL_LABELS), 1)
    lout_ref[0] = (iota_l == final_lab).astype(jnp.float32)

    aux_ref[0] = jnp.concatenate(
        [d_y, d_x, d_h, d_w, final_lab.astype(jnp.float32)], axis=1)


def _stage_b(aux_ref, oy_ref, ox_ref, oh_ref, ow_ref):
    aux = aux_ref[0]
    fl = aux[:, 4:5].astype(jnp.int32)
    iota_l = lax.broadcasted_iota(jnp.int32, (_CHUNK, _TOTAL_LABELS), 1)
    onehot = (iota_l == fl).astype(jnp.float32)
    oy_ref[0] = onehot * aux[:, 0:1]
    ox_ref[0] = onehot * aux[:, 1:2]
    oh_ref[0] = onehot * aux[:, 2:3]
    ow_ref[0] = onehot * aux[:, 3:4]


def _run(pk, gtt, lab3, interpret=False):
    b = pk.shape[0]
    labels, aux = pl.pallas_call(
        _stage_a,
        grid=(b,),
        in_specs=[
            pl.BlockSpec((1, _N_ROI, 6), lambda i: (i, 0, 0)),
            pl.BlockSpec((1, 4, _N_GT), lambda i: (i, 0, 0)),
            pl.BlockSpec((1, 1, _N_GT), lambda i: (i, 0, 0)),
        ],
        out_specs=[
            pl.BlockSpec((1, _N_ROI, _TOTAL_LABELS), lambda i: (i, 0, 0)),
            pl.BlockSpec((1, _N_ROI, 5), lambda i: (i, 0, 0)),
        ],
        out_shape=[
            jax.ShapeDtypeStruct((b, _N_ROI, _TOTAL_LABELS), jnp.float32),
            jax.ShapeDtypeStruct((b, _N_ROI, 5), jnp.float32),
        ],
        interpret=interpret,
    )(pk, gtt, lab3)

    comp_shape = jax.ShapeDtypeStruct((b, _N_ROI, _TOTAL_LABELS), jnp.float32)
    comp_spec = pl.BlockSpec((1, _CHUNK, _TOTAL_LABELS), lambda i, j: (i, j, 0))
    oy, ox, oh, ow = pl.pallas_call(
        _stage_b,
        grid=(b, _N_ROI // _CHUNK),
        in_specs=[pl.BlockSpec((1, _CHUNK, 5), lambda i, j: (i, j, 0))],
        out_specs=[comp_spec, comp_spec, comp_spec, comp_spec],
        out_shape=[comp_shape, comp_shape, comp_shape, comp_shape],
        interpret=interpret,
    )(aux)
    return (oy, ox, oh, ow), labels


def kernel(roi_bboxes, gt_boxes, gt_labels):
    b, n = roi_bboxes.shape[0], roi_bboxes.shape[1]
    gtt = jnp.transpose(gt_boxes, (0, 2, 1))
    lab3 = gt_labels[:, None, :]
    rpos = jax.random.randint(jax.random.key(_SEED), (b, n), 1,
                              _TOTAL_POS * 10, dtype=jnp.int32)[..., None]
    rneg = jax.random.randint(jax.random.key(_SEED + 1), (b, n), 1,
                              _TOTAL_NEG * 10, dtype=jnp.int32)[..., None]
    pk = jnp.concatenate(
        [roi_bboxes, rpos.astype(jnp.float32), rneg.astype(jnp.float32)],
        axis=-1)
    (oy, ox, oh, ow), labels = _run(pk, gtt, lab3)
    deltas = jnp.stack([oy, ox, oh, ow], axis=-1).reshape(b, n * _TOTAL_LABELS, 4)
    return deltas, labels
